# baseline (speedup 1.0000x reference)
import functools

import jax
import jax.numpy as jnp
from jax import lax
from jax.experimental import pallas as pl
from jax.experimental.pallas import tpu as pltpu
from jax.experimental.pallas import tpu_sc as plsc

VOCAB = 8192
BATCH = 4
SEQ = 2048
N_ROWS = BATCH * SEQ
NC, NS = 2, 16
NW = NC * NS
ROWS_PER_W = N_ROWS // NW
CHUNK = 4
N_CHUNKS = ROWS_PER_W // CHUNK
NBUF = 2


def _make_gather():
    mesh = plsc.VectorSubcoreMesh(core_axis_name="c", subcore_axis_name="s")

    @functools.partial(
        pl.kernel,
        out_type=jax.ShapeDtypeStruct((N_ROWS, VOCAB), jnp.float32),
        mesh=mesh,
        scratch_types=[
            pltpu.VMEM((N_CHUNKS, CHUNK), jnp.int32),
            pltpu.VMEM((CHUNK, VOCAB), jnp.float32),
            pltpu.VMEM((CHUNK, VOCAB), jnp.float32),
            pltpu.SemaphoreType.DMA,
            pltpu.SemaphoreType.DMA,
            pltpu.SemaphoreType.DMA,
            pltpu.SemaphoreType.DMA,
        ],
    )
    def gather_kernel(x_hbm, w_hbm, out_hbm, idx_v, rows0, rows1,
                      gsem0, gsem1, osem0, osem1):
        rows = (rows0, rows1)
        gsem = (gsem0, gsem1)
        osem = (osem0, osem1)
        wid = lax.axis_index("s") * NC + lax.axis_index("c")
        base = wid * N_CHUNKS
        pltpu.sync_copy(x_hbm.at[pl.ds(base, N_CHUNKS)], idx_v)

        for b in range(NBUF):
            pltpu.async_copy(w_hbm.at[idx_v.at[b]], rows[b], gsem[b])

        def body(i, carry):
            g = i * NBUF
            for b in range(NBUF):
                c = g + b
                pltpu.make_async_copy(
                    w_hbm.at[idx_v.at[b]], rows[b], gsem[b]
                ).wait()
                pltpu.async_copy(
                    rows[b], out_hbm.at[pl.ds((base + c) * CHUNK, CHUNK)],
                    osem[b],
                ).wait()
                @pl.when(c + NBUF < N_CHUNKS)
                def _():
                    pltpu.async_copy(
                        w_hbm.at[idx_v.at[c + NBUF]], rows[b], gsem[b]
                    )
            return carry

        lax.fori_loop(0, N_CHUNKS // NBUF, body, 0)

    return gather_kernel


_gather = _make_gather()


def kernel(x, W):
    x2 = x.reshape(N_ROWS // CHUNK, CHUNK).astype(jnp.int32)
    out = _gather(x2, W)
    return out.reshape(BATCH, SEQ, VOCAB)
